# baseline (speedup 1.0000x reference)
import functools

import jax
import jax.numpy as jnp
from jax import lax
from jax.experimental import pallas as pl
from jax.experimental.pallas import tpu as pltpu, tpu_sc as plsc


def _tc_copy(kin, kout):
    kout[...] = kin[...]


def _tc_scatter(pos_ref, val, copy_hbm, out):
    del pos_ref, copy_hbm
    out[...] = val[...]


def _sc_copy(vcache_hbm, vout_hbm, stage_v, sem_ld, sem_st, *, nw, rows_per):
    wid = lax.axis_index("c") * (nw // 2) + lax.axis_index("s")
    base = wid * rows_per

    ch = stage_v[0].shape[0]
    n_ch = rows_per // ch
    nbuf = len(stage_v)
    ld_d = [None] * n_ch
    st_d = [None] * n_ch

    for i in range(n_ch):
        b = i % nbuf
        if i >= nbuf:
            st_d[i - nbuf].wait()
        off = base + i * ch
        ld_d[i] = pltpu.async_copy(vcache_hbm.at[pl.ds(off, ch)], stage_v[b],
                                   sem_ld[b])
        j = i - 2
        if j >= 0:
            ld_d[j].wait()
            offj = base + j * ch
            st_d[j] = pltpu.async_copy(stage_v[j % nbuf],
                                       vout_hbm.at[pl.ds(offj, ch)],
                                       sem_st[j % nbuf])
    for j in (n_ch - 2, n_ch - 1):
        ld_d[j].wait()
        offj = base + j * ch
        st_d[j] = pltpu.async_copy(stage_v[j % nbuf],
                                   vout_hbm.at[pl.ds(offj, ch)],
                                   sem_st[j % nbuf])
    for d_ in st_d[-nbuf:]:
        d_.wait()


def _scatter_call(pos, val4, copy4, bh, s, s_max, d, dtype):
    grid_spec = pltpu.PrefetchScalarGridSpec(
        num_scalar_prefetch=1,
        grid=(s,),
        in_specs=[
            pl.BlockSpec((bh, 1, 1, d), lambda j, pos_ref: (0, j, 0, 0)),
            pl.BlockSpec(memory_space=pltpu.HBM),
        ],
        out_specs=pl.BlockSpec((bh, 1, 1, d),
                               lambda j, pos_ref: (0, pos_ref[j], 0, 0)),
    )
    return pl.pallas_call(
        _tc_scatter,
        grid_spec=grid_spec,
        out_shape=jax.ShapeDtypeStruct((bh, s_max, 1, d), dtype),
        input_output_aliases={2: 0},
    )(pos, val4, copy4)


def kernel(input_pos, k_val, v_val, k_cache, v_cache):
    b, h, s_max, d = k_cache.shape
    s = k_val.shape[2]
    bh = b * h
    total_rows = bh * s_max

    mesh = plsc.VectorSubcoreMesh(core_axis_name="c", subcore_axis_name="s")
    nw = mesh.num_cores * mesh.num_subcores
    assert total_rows % nw == 0
    rows_per = total_rows // nw

    pos = input_pos.astype(jnp.int32)

    nblk = 32
    k_copy = pl.pallas_call(
        _tc_copy,
        grid=(nblk,),
        in_specs=[pl.BlockSpec((total_rows // nblk, d), lambda i: (i, 0))],
        out_specs=pl.BlockSpec((total_rows // nblk, d), lambda i: (i, 0)),
        out_shape=jax.ShapeDtypeStruct((total_rows, d), k_cache.dtype),
    )(k_cache.reshape(total_rows, d))

    v_copy = pl.kernel(
        functools.partial(_sc_copy, nw=nw, rows_per=rows_per),
        out_type=jax.ShapeDtypeStruct((total_rows, d), v_cache.dtype),
        mesh=mesh,
        scratch_types=[
            [pltpu.VMEM((128, d), jnp.float32) for _ in range(4)],
            [pltpu.SemaphoreType.DMA for _ in range(4)],
            [pltpu.SemaphoreType.DMA for _ in range(4)],
        ],
    )(v_cache.reshape(total_rows, d))

    k_out = _scatter_call(pos, k_val.reshape(bh, s, 1, d),
                          k_copy.reshape(bh, s_max, 1, d),
                          bh, s, s_max, d, k_cache.dtype)
    v_out = _scatter_call(pos, v_val.reshape(bh, s, 1, d),
                          v_copy.reshape(bh, s_max, 1, d),
                          bh, s, s_max, d, v_cache.dtype)
    return (k_out.reshape(b, h, s_max, d), v_out.reshape(b, h, s_max, d))
